# Initial kernel scaffold; baseline (speedup 1.0000x reference)
import jax
import jax.numpy as jnp
from jax.experimental import pallas as pl


def kernel(x, edge_index, W_l0, b_l0, W_r0, W_l1, b_l1, W_r1):
    raise NotImplementedError("write your pallas kernel here")


# speedup vs baseline: 4.3697x; 4.3697x over previous
import functools

import jax
import jax.numpy as jnp
from jax import lax
from jax.experimental import pallas as pl
from jax.experimental.pallas import tpu as pltpu
from jax.experimental.pallas import tpu_sc as plsc

N = 10000
E = 160000
D = 256
H = 128
NC = 2
NS = 16
CH = 128
NCHUNK = E // CH
CHUNKS_PER_TILE = -(-NCHUNK // NS)
RPT = 624
TAIL = N - NS * RPT
ZR = 104

_MESH = plsc.VectorSubcoreMesh(core_axis_name="c", subcore_axis_name="s",
                               num_cores=NC, num_subcores=NS)


def _sc_agg_body(t_hbm, src2_hbm, dst_hbm, agg_hbm, srcv, dstv, rows,
                 acc, sem):
    c = lax.axis_index("c")
    s = lax.axis_index("s")

    z16 = jnp.zeros((16,), jnp.float32)

    def zfill(r, _):
        for k in range(H // 16):
            rows[r, pl.ds(k * 16, 16)] = z16
        return 0
    lax.fori_loop(0, ZR, zfill, 0)
    for j in range(RPT // ZR):
        pltpu.sync_copy(rows.at[pl.ds(0, ZR)],
                        acc.at[pl.ds(s * RPT + j * ZR, ZR)])

    @pl.when(s == 0)
    def _():
        pltpu.sync_copy(rows.at[pl.ds(0, TAIL)],
                        acc.at[pl.ds(NS * RPT, TAIL)])

    plsc.subcore_barrier()

    def chunk(i, _):
        ch = i * NS + s

        @pl.when(ch < NCHUNK)
        def _():
            base = ch * CH
            pltpu.sync_copy(src2_hbm.at[pl.ds(c * E + base, CH)], srcv)
            pltpu.sync_copy(dst_hbm.at[pl.ds(base, CH)], dstv)
            pltpu.async_copy(t_hbm.at[srcv], rows, sem).wait()
            pltpu.sync_copy(rows, acc.at[dstv], add=True)
        return 0
    lax.fori_loop(0, CHUNKS_PER_TILE, chunk, 0)

    plsc.subcore_barrier()

    pltpu.sync_copy(acc.at[pl.ds(s * RPT, RPT)],
                    agg_hbm.at[pl.ds(c * N + s * RPT, RPT)])

    @pl.when(s == 0)
    def _():
        pltpu.sync_copy(acc.at[pl.ds(NS * RPT, TAIL)],
                        agg_hbm.at[pl.ds(c * N + NS * RPT, TAIL)])


_sc_agg = pl.kernel(
    _sc_agg_body,
    out_type=jax.ShapeDtypeStruct((NC * N, H), jnp.float32),
    mesh=_MESH,
    scratch_types=[
        pltpu.VMEM((CH,), jnp.int32),
        pltpu.VMEM((CH,), jnp.int32),
        pltpu.VMEM((CH, H), jnp.float32),
        pltpu.VMEM_SHARED((N, H), jnp.float32),
        pltpu.SemaphoreType.DMA,
    ],
)

DW = 128
DEG_ITERS = -(-NCHUNK // (NC * NS))


def _sc_deg_body(dst_hbm, deg_hbm, dstv, ones, degacc):
    c = lax.axis_index("c")
    s = lax.axis_index("s")
    w = s * NC + c

    one16 = jnp.full((16,), 1.0, jnp.float32)
    z16 = jnp.zeros((16,), jnp.float32)

    def zfill(r, _):
        for k in range(DW // 16):
            ones[r, pl.ds(k * 16, 16)] = z16
        return 0
    lax.fori_loop(0, ZR, zfill, 0)
    for j in range(RPT // ZR):
        pltpu.sync_copy(ones.at[pl.ds(0, ZR)],
                        degacc.at[pl.ds(s * RPT + j * ZR, ZR)])

    @pl.when(s == 0)
    def _():
        pltpu.sync_copy(ones.at[pl.ds(0, TAIL)],
                        degacc.at[pl.ds(NS * RPT, TAIL)])

    def ofill(r, _):
        for k in range(DW // 16):
            ones[r, pl.ds(k * 16, 16)] = one16
        return 0
    lax.fori_loop(0, CH, ofill, 0)

    plsc.subcore_barrier()

    def chunk(i, _):
        ch = i * (NC * NS) + w

        @pl.when(ch < NCHUNK)
        def _():
            pltpu.sync_copy(dst_hbm.at[pl.ds(ch * CH, CH)], dstv)
            pltpu.sync_copy(ones, degacc.at[dstv], add=True)
        return 0
    lax.fori_loop(0, DEG_ITERS, chunk, 0)

    plsc.subcore_barrier()

    pltpu.sync_copy(degacc.at[pl.ds(s * RPT, RPT)],
                    deg_hbm.at[pl.ds(c * N + s * RPT, RPT)])

    @pl.when(s == 0)
    def _():
        pltpu.sync_copy(degacc.at[pl.ds(NS * RPT, TAIL)],
                        deg_hbm.at[pl.ds(c * N + NS * RPT, TAIL)])


_sc_deg = pl.kernel(
    _sc_deg_body,
    out_type=jax.ShapeDtypeStruct((NC * N, DW), jnp.float32),
    mesh=_MESH,
    scratch_types=[
        pltpu.VMEM((CH,), jnp.int32),
        pltpu.VMEM((CH, DW), jnp.float32),
        pltpu.VMEM_SHARED((N, DW), jnp.float32),
    ],
)


def _dot(a, b):
    return jnp.dot(a, b, preferred_element_type=jnp.float32,
                   precision=lax.Precision.HIGHEST)


_RB = 1000
_GRID = N // _RB


def _tc_lin0_body(x_ref, wl_ref, wr_ref, t_ref, r_ref):
    xb = x_ref[...]
    y = _dot(xb, wl_ref[...])
    t_ref[0] = y[:, :H]
    t_ref[1] = y[:, H:]
    r_ref[...] = _dot(xb, wr_ref[...])


def _tc_lin0(x, W_l, W_r):
    return pl.pallas_call(
        _tc_lin0_body,
        grid=(_GRID,),
        in_specs=[
            pl.BlockSpec((_RB, D), lambda i: (i, 0)),
            pl.BlockSpec((D, D), lambda i: (0, 0)),
            pl.BlockSpec((D, D), lambda i: (0, 0)),
        ],
        out_specs=[
            pl.BlockSpec((NC, _RB, H), lambda i: (0, i, 0)),
            pl.BlockSpec((_RB, D), lambda i: (i, 0)),
        ],
        out_shape=[
            jax.ShapeDtypeStruct((NC, N, H), jnp.float32),
            jax.ShapeDtypeStruct((N, D), jnp.float32),
        ],
    )(x, W_l, W_r)


def _tc_mid_body(agg_ref, deg_ref, b_ref, r_ref, wl_ref, wr_ref,
                 t_ref, r1_ref):
    deg = jnp.maximum(deg_ref[0, :, 0:1] + deg_ref[1, :, 0:1], 1.0)
    y = jnp.concatenate([agg_ref[0], agg_ref[1]], axis=1)
    h = jnp.maximum(y / deg + b_ref[...] + r_ref[...], 0.0)
    y1 = _dot(h, wl_ref[...])
    t_ref[0] = y1[:, :H]
    t_ref[1] = y1[:, H:]
    r1_ref[...] = _dot(h, wr_ref[...])


def _tc_mid(agg, degt, b, r, W_l, W_r):
    return pl.pallas_call(
        _tc_mid_body,
        grid=(_GRID,),
        in_specs=[
            pl.BlockSpec((NC, _RB, H), lambda i: (0, i, 0)),
            pl.BlockSpec((NC, _RB, DW), lambda i: (0, i, 0)),
            pl.BlockSpec((1, D), lambda i: (0, 0)),
            pl.BlockSpec((_RB, D), lambda i: (i, 0)),
            pl.BlockSpec((D, D), lambda i: (0, 0)),
            pl.BlockSpec((D, D), lambda i: (0, 0)),
        ],
        out_specs=[
            pl.BlockSpec((NC, _RB, H), lambda i: (0, i, 0)),
            pl.BlockSpec((_RB, D), lambda i: (i, 0)),
        ],
        out_shape=[
            jax.ShapeDtypeStruct((NC, N, H), jnp.float32),
            jax.ShapeDtypeStruct((N, D), jnp.float32),
        ],
    )(agg, degt, b, r, W_l, W_r)


def _tc_out_body(agg_ref, deg_ref, b_ref, r_ref, o_ref):
    deg = jnp.maximum(deg_ref[0, :, 0:1] + deg_ref[1, :, 0:1], 1.0)
    y = jnp.concatenate([agg_ref[0], agg_ref[1]], axis=1)
    y = y / deg + b_ref[...] + r_ref[...]
    m = jnp.max(y, axis=1, keepdims=True)
    lse = m + jnp.log(jnp.sum(jnp.exp(y - m), axis=1, keepdims=True))
    o_ref[...] = y - lse


def _tc_out(agg, degt, b, r):
    return pl.pallas_call(
        _tc_out_body,
        grid=(_GRID,),
        in_specs=[
            pl.BlockSpec((NC, _RB, H), lambda i: (0, i, 0)),
            pl.BlockSpec((NC, _RB, DW), lambda i: (0, i, 0)),
            pl.BlockSpec((1, D), lambda i: (0, 0)),
            pl.BlockSpec((_RB, D), lambda i: (i, 0)),
        ],
        out_specs=pl.BlockSpec((_RB, D), lambda i: (i, 0)),
        out_shape=jax.ShapeDtypeStruct((N, D), jnp.float32),
    )(agg, degt, b, r)


def kernel(x, edge_index, W_l0, b_l0, W_r0, W_l1, b_l1, W_r1):
    src = edge_index[0].astype(jnp.int32)
    dst = edge_index[1].astype(jnp.int32)
    src2 = jnp.concatenate([src, src + N])

    t0, r0 = _tc_lin0(x, W_l0, W_r0)
    degp = _sc_deg(dst).reshape(NC, N, DW)
    agg0 = _sc_agg(t0.reshape(NC * N, H), src2, dst)
    t1, r1 = _tc_mid(agg0.reshape(NC, N, H), degp, b_l0.reshape(1, D), r0,
                     W_l1, W_r1)
    agg1 = _sc_agg(t1.reshape(NC * N, H), src2, dst)
    return _tc_out(agg1.reshape(NC, N, H), degp, b_l1.reshape(1, D), r1)
